# Initial kernel scaffold; baseline (speedup 1.0000x reference)
import jax
import jax.numpy as jnp
from jax.experimental import pallas as pl


def kernel(pred_x, pred_q, target_x, target_q, edge2graph, node2graph, atom_type, edge_r, edge_p):
    raise NotImplementedError("write your pallas kernel here")


# speedup vs baseline: 10.9405x; 10.9405x over previous
import functools

import jax
import jax.numpy as jnp
from jax import lax
from jax.experimental import pallas as pl
from jax.experimental.pallas import tpu as pltpu
from jax.experimental.pallas import tpu_sc as plsc

NUM_GRAPHS = 256
NC = 2
NS = 16
NW = NC * NS
L = 16

ACCW = 384
RED = 3 * ACCW

EB = 4000


def _sc_body(pxf, txf, n2g, pq, tq, e2g,
             out,
             p_buf, t_buf, g_buf,
             nx_buf, ntx_buf, ng_buf,
             acc_x, acc_c, acc_q,
             red_buf, big_buf, res_buf,
             shared,
             sem0, sem1,
             *, e_per_w, n_per_w):
    c = lax.axis_index("c")
    s = lax.axis_index("s")
    wid = c * NS + s
    iota = lax.iota(jnp.int32, L)
    lane_base = iota * ACCW
    zeros = jnp.zeros((L,), jnp.float32)
    ones = jnp.ones((L,), jnp.float32)
    f_per_w = 3 * n_per_w
    n_blk = e_per_w // EB
    sems = (sem0, sem1)

    eb_base = wid * e_per_w

    def start_block(b):
        slot = b % 2
        base = eb_base + b * EB
        return (
            pltpu.async_copy(pq.at[pl.ds(base, EB)],
                             p_buf.at[pl.ds(slot * EB, EB)], sems[slot]),
            pltpu.async_copy(tq.at[pl.ds(base, EB)],
                             t_buf.at[pl.ds(slot * EB, EB)], sems[slot]),
            pltpu.async_copy(e2g.at[pl.ds(base, EB)],
                             g_buf.at[pl.ds(slot * EB, EB)], sems[slot]),
        )

    inflight = {0: start_block(0)}

    def zbody(j, _):
        acc_x[pl.ds(j * L, L)] = zeros
        acc_c[pl.ds(j * L, L)] = zeros
        acc_q[pl.ds(j * L, L)] = zeros
        return 0
    lax.fori_loop(0, (L * ACCW) // L, zbody, 0)

    nb = wid * f_per_w
    pltpu.sync_copy(pxf.at[pl.ds(nb, f_per_w)], nx_buf)
    pltpu.sync_copy(txf.at[pl.ds(nb, f_per_w)], ntx_buf)
    pltpu.sync_copy(n2g.at[pl.ds(wid * n_per_w, n_per_w)], ng_buf)

    def cbody(v, _):
        g = ng_buf[pl.ds(v * L, L)]
        plsc.addupdate_scatter(acc_c, [lane_base + g], ones)
        return 0
    lax.fori_loop(0, n_per_w // L, cbody, 0)

    def xbody(v, _):
        off = v * L
        p = nx_buf[pl.ds(off, L)]
        t = ntx_buf[pl.ds(off, L)]
        d = p - t
        nid = (off + iota) // 3
        g = plsc.load_gather(ng_buf, [nid])
        plsc.addupdate_scatter(acc_x, [lane_base + g], d * d)
        return 0
    lax.fori_loop(0, f_per_w // L, xbody, 0)

    for b in range(n_blk):
        slot = b % 2
        if b + 1 < n_blk:
            inflight[b + 1] = start_block(b + 1)
        for h in inflight.pop(b):
            h.wait()
        sbase = slot * EB

        def ebody(v, _):
            off = sbase + v * L
            p = p_buf[pl.ds(off, L)]
            t = t_buf[pl.ds(off, L)]
            g = g_buf[pl.ds(off, L)]
            d = p - t
            plsc.addupdate_scatter(acc_q, [lane_base + g], d * d)
            return 0
        lax.fori_loop(0, EB // L, ebody, 0)

    def rbody(j, _):
        col = j * L
        sx = zeros
        sc = zeros
        sq = zeros
        for i in range(L):
            sx = sx + acc_x[pl.ds(i * ACCW + col, L)]
            sc = sc + acc_c[pl.ds(i * ACCW + col, L)]
            sq = sq + acc_q[pl.ds(i * ACCW + col, L)]
        red_buf[pl.ds(col, L)] = sx
        red_buf[pl.ds(ACCW + col, L)] = sc
        red_buf[pl.ds(2 * ACCW + col, L)] = sq
        return 0
    lax.fori_loop(0, ACCW // L, rbody, 0)

    pltpu.sync_copy(red_buf, shared.at[pl.ds(s * RED, RED)])
    plsc.subcore_barrier()

    @pl.when(s == 0)
    def _():
        pltpu.sync_copy(shared, big_buf)

        def fbody(j, _):
            col = j * L
            acc = zeros
            for i in range(NS):
                acc = acc + big_buf[pl.ds(i * RED + col, L)]
            res_buf[pl.ds(col, L)] = acc
            return 0
        lax.fori_loop(0, RED // L, fbody, 0)
        pltpu.sync_copy(res_buf, out.at[pl.ds(c * RED, RED)])


def _fin_body(p_ref, o1_ref, o2_ref):
    p = p_ref[...]
    t = p[0] + p[1]
    sx = t[0:1, :]
    cnt = t[1:2, :]
    sq = t[2:3, :]
    rmsd = jnp.sqrt(sx / jnp.maximum(cnt, 1.0))
    o1_ref[0, 0] = jnp.sum(rmsd) * (1.0 / NUM_GRAPHS)
    o2_ref[0, 0] = jnp.sum(jnp.sqrt(sq)) * (1.0 / NUM_GRAPHS)


def kernel(pred_x, pred_q, target_x, target_q, edge2graph, node2graph,
           atom_type, edge_r, edge_p):
    n = node2graph.shape[0]
    e = edge2graph.shape[0]
    assert e % (NW * L) == 0 and (e // NW) % EB == 0
    e_per_w = e // NW

    n_per_w = -(-n // NW)
    n_per_w = ((n_per_w + L - 1) // L) * L
    n_pad = n_per_w * NW
    pxf = jnp.pad(pred_x, ((0, n_pad - n), (0, 0))).reshape(-1)
    txf = jnp.pad(target_x, ((0, n_pad - n), (0, 0))).reshape(-1)
    n2gp = jnp.pad(node2graph, (0, n_pad - n), constant_values=NUM_GRAPHS)

    sc_call = pl.kernel(
        functools.partial(_sc_body, e_per_w=e_per_w, n_per_w=n_per_w),
        out_type=jax.ShapeDtypeStruct((NC * RED,), jnp.float32),
        mesh=plsc.VectorSubcoreMesh(core_axis_name="c", subcore_axis_name="s"),
        compiler_params=pltpu.CompilerParams(needs_layout_passes=False),
        scratch_types=[
            pltpu.VMEM((2 * EB,), jnp.float32),
            pltpu.VMEM((2 * EB,), jnp.float32),
            pltpu.VMEM((2 * EB,), jnp.int32),
            pltpu.VMEM((3 * n_per_w,), jnp.float32),
            pltpu.VMEM((3 * n_per_w,), jnp.float32),
            pltpu.VMEM((n_per_w,), jnp.int32),
            pltpu.VMEM((L * ACCW,), jnp.float32),
            pltpu.VMEM((L * ACCW,), jnp.float32),
            pltpu.VMEM((L * ACCW,), jnp.float32),
            pltpu.VMEM((RED,), jnp.float32),
            pltpu.VMEM((NS * RED,), jnp.float32),
            pltpu.VMEM((RED,), jnp.float32),
            pltpu.VMEM_SHARED((NS * RED,), jnp.float32),
            pltpu.SemaphoreType.DMA,
            pltpu.SemaphoreType.DMA,
        ],
    )
    partials = sc_call(pxf, txf, n2gp, pred_q, target_q, edge2graph)
    partials = partials.reshape(NC, 3, ACCW)

    r1, r2 = pl.pallas_call(
        _fin_body,
        out_shape=(jax.ShapeDtypeStruct((1, 1), jnp.float32),
                   jax.ShapeDtypeStruct((1, 1), jnp.float32)),
        in_specs=[pl.BlockSpec(memory_space=pltpu.VMEM)],
        out_specs=(pl.BlockSpec(memory_space=pltpu.SMEM),
                   pl.BlockSpec(memory_space=pltpu.SMEM)),
    )(partials)
    return (r1[0, 0], r2[0, 0])


# speedup vs baseline: 13.9279x; 1.2731x over previous
import functools

import jax
import jax.numpy as jnp
from jax import lax
from jax.experimental import pallas as pl
from jax.experimental.pallas import tpu as pltpu
from jax.experimental.pallas import tpu_sc as plsc

NUM_GRAPHS = 256
NC = 2
NS = 16
NW = NC * NS
L = 16

ACCW = 385
REDW = 384
RED = 3 * REDW

EB = 4000
EU = 5


def _sc_body(pxh, txh, n2g, pq, tq, e2g,
             out,
             p_buf, t_buf, g_buf,
             nx_buf, ntx_buf, ng_buf,
             acc_x, acc_c, acc_q,
             red_buf, big_buf, res_buf,
             shared,
             sem0, sem1,
             *, e_per_w, n_main, n_last):
    c = lax.axis_index("c")
    s = lax.axis_index("s")
    wid = c * NS + s
    iota = lax.iota(jnp.int32, L)
    lane_base = iota * ACCW
    zeros = jnp.zeros((L,), jnp.float32)
    ones = jnp.ones((L,), jnp.float32)
    col0 = jnp.zeros((L,), jnp.int32)
    col1 = jnp.ones((L,), jnp.int32)
    col2 = jnp.full((L,), 2, jnp.int32)
    n_blk = e_per_w // EB
    sems = (sem0, sem1)

    eb_base = wid * e_per_w

    def start_block(b):
        slot = b % 2
        base = eb_base + b * EB
        return (
            pltpu.async_copy(pq.at[pl.ds(base, EB)],
                             p_buf.at[pl.ds(slot * EB, EB)], sems[slot]),
            pltpu.async_copy(tq.at[pl.ds(base, EB)],
                             t_buf.at[pl.ds(slot * EB, EB)], sems[slot]),
            pltpu.async_copy(e2g.at[pl.ds(base, EB)],
                             g_buf.at[pl.ds(slot * EB, EB)], sems[slot]),
        )

    inflight = {0: start_block(0)}

    def zbody(j, _):
        acc_x[pl.ds(j * L, L)] = zeros
        acc_c[pl.ds(j * L, L)] = zeros
        acc_q[pl.ds(j * L, L)] = zeros
        return 0
    lax.fori_loop(0, (L * ACCW) // L, zbody, 0)

    def node_part(nrows):
        rbase = wid * n_main
        pltpu.sync_copy(pxh.at[pl.ds(rbase, nrows)],
                        nx_buf.at[pl.ds(0, nrows)])
        pltpu.sync_copy(txh.at[pl.ds(rbase, nrows)],
                        ntx_buf.at[pl.ds(0, nrows)])
        pltpu.sync_copy(n2g.at[pl.ds(rbase, nrows)],
                        ng_buf.at[pl.ds(0, nrows)])

        def nbody(v, _):
            rows = v * L + iota
            ids = ng_buf[pl.ds(v * L, L)]
            p0 = plsc.load_gather(nx_buf, [rows, col0])
            p1 = plsc.load_gather(nx_buf, [rows, col1])
            p2 = plsc.load_gather(nx_buf, [rows, col2])
            t0 = plsc.load_gather(ntx_buf, [rows, col0])
            t1 = plsc.load_gather(ntx_buf, [rows, col1])
            t2 = plsc.load_gather(ntx_buf, [rows, col2])
            d0 = p0 - t0
            d1 = p1 - t1
            d2 = p2 - t2
            e2 = d0 * d0 + d1 * d1 + d2 * d2
            idx = lane_base + ids
            plsc.addupdate_scatter(acc_x, [idx], e2)
            plsc.addupdate_scatter(acc_c, [idx], ones)
            return 0
        lax.fori_loop(0, nrows // L, nbody, 0)

    @pl.when(wid < NW - 1)
    def _():
        node_part(n_main)

    @pl.when(wid == NW - 1)
    def _():
        node_part(n_last)

    for b in range(n_blk):
        slot = b % 2
        if b + 1 < n_blk:
            inflight[b + 1] = start_block(b + 1)
        for h in inflight.pop(b):
            h.wait()
        sbase = slot * EB

        def ebody(v, _):
            off = sbase + v * (EU * L)
            for u in range(EU):
                o = off + u * L
                p = p_buf[pl.ds(o, L)]
                t = t_buf[pl.ds(o, L)]
                g = g_buf[pl.ds(o, L)]
                d = p - t
                plsc.addupdate_scatter(acc_q, [lane_base + g], d * d)
            return 0
        lax.fori_loop(0, EB // (EU * L), ebody, 0)

    def rbody(j, _):
        col = j * L
        sx = zeros
        sc = zeros
        sq = zeros
        for i in range(L):
            gidx = i * ACCW + col + iota
            sx = sx + plsc.load_gather(acc_x, [gidx])
            sc = sc + plsc.load_gather(acc_c, [gidx])
            sq = sq + plsc.load_gather(acc_q, [gidx])
        red_buf[pl.ds(col, L)] = sx
        red_buf[pl.ds(REDW + col, L)] = sc
        red_buf[pl.ds(2 * REDW + col, L)] = sq
        return 0
    lax.fori_loop(0, REDW // L, rbody, 0)

    pltpu.sync_copy(red_buf, shared.at[pl.ds(s * RED, RED)])
    plsc.subcore_barrier()

    @pl.when(s == 0)
    def _():
        pltpu.sync_copy(shared, big_buf)

        def fbody(j, _):
            col = j * L
            acc = zeros
            for i in range(NS):
                acc = acc + big_buf[pl.ds(i * RED + col, L)]
            res_buf[pl.ds(col, L)] = acc
            return 0
        lax.fori_loop(0, RED // L, fbody, 0)
        pltpu.sync_copy(res_buf, out.at[pl.ds(c * RED, RED)])


def _fin_body(p_ref, o1_ref, o2_ref):
    p = p_ref[...]
    t = p[0] + p[1]
    sx = t[0:1, :]
    cnt = t[1:2, :]
    sq = t[2:3, :]
    rmsd = jnp.sqrt(sx / jnp.maximum(cnt, 1.0))
    o1_ref[0, 0] = jnp.sum(rmsd) * (1.0 / NUM_GRAPHS)
    o2_ref[0, 0] = jnp.sum(jnp.sqrt(sq)) * (1.0 / NUM_GRAPHS)


def kernel(pred_x, pred_q, target_x, target_q, edge2graph, node2graph,
           atom_type, edge_r, edge_p):
    n = node2graph.shape[0]
    e = edge2graph.shape[0]
    assert e % (NW * L) == 0 and (e // NW) % EB == 0
    e_per_w = e // NW

    n_main = ((-(-n // NW) + L - 1) // L) * L
    n_last = n - (NW - 1) * n_main
    assert 0 < n_last <= n_main and n_last % L == 0 and n_main % 8 == 0

    sc_call = pl.kernel(
        functools.partial(_sc_body, e_per_w=e_per_w,
                          n_main=n_main, n_last=n_last),
        out_type=jax.ShapeDtypeStruct((NC * RED,), jnp.float32),
        mesh=plsc.VectorSubcoreMesh(core_axis_name="c", subcore_axis_name="s"),
        compiler_params=pltpu.CompilerParams(needs_layout_passes=False,
                                             use_tc_tiling_on_sc=False),
        scratch_types=[
            pltpu.VMEM((2 * EB,), jnp.float32),
            pltpu.VMEM((2 * EB,), jnp.float32),
            pltpu.VMEM((2 * EB,), jnp.int32),
            pltpu.VMEM((n_main, 3), jnp.float32),
            pltpu.VMEM((n_main, 3), jnp.float32),
            pltpu.VMEM((n_main,), jnp.int32),
            pltpu.VMEM((L * ACCW,), jnp.float32),
            pltpu.VMEM((L * ACCW,), jnp.float32),
            pltpu.VMEM((L * ACCW,), jnp.float32),
            pltpu.VMEM((RED,), jnp.float32),
            pltpu.VMEM((NS * RED,), jnp.float32),
            pltpu.VMEM((RED,), jnp.float32),
            pltpu.VMEM_SHARED((NS * RED,), jnp.float32),
            pltpu.SemaphoreType.DMA,
            pltpu.SemaphoreType.DMA,
        ],
    )
    partials = sc_call(pred_x, target_x, node2graph,
                       pred_q, target_q, edge2graph)
    partials = partials.reshape(NC, 3, REDW)

    r1, r2 = pl.pallas_call(
        _fin_body,
        out_shape=(jax.ShapeDtypeStruct((1, 1), jnp.float32),
                   jax.ShapeDtypeStruct((1, 1), jnp.float32)),
        in_specs=[pl.BlockSpec(memory_space=pltpu.VMEM)],
        out_specs=(pl.BlockSpec(memory_space=pltpu.SMEM),
                   pl.BlockSpec(memory_space=pltpu.SMEM)),
    )(partials)
    return (r1[0, 0], r2[0, 0])


# speedup vs baseline: 22.7561x; 1.6339x over previous
import functools

import jax
import jax.numpy as jnp
from jax import lax
from jax.experimental import pallas as pl
from jax.experimental.pallas import tpu as pltpu
from jax.experimental.pallas import tpu_sc as plsc

NUM_GRAPHS = 256
NC = 2
NS = 16
NW = NC * NS
L = 16

ACCW = 385
REDW = 384
RED = 3 * REDW

EB = 10000
EU = 5

NR = 3200


def _err_body(px_ref, tx_ref, o_ref):
    d = px_ref[...] - tx_ref[...]
    o_ref[...] = lax.dot_general(
        jnp.ones((1, 3), jnp.float32), d * d,
        (((1,), (1,)), ((), ())),
        preferred_element_type=jnp.float32)


def _sc_body(err, n2g, pq, tq, e2g,
             out,
             p_buf, t_buf, g_buf,
             ne_buf, ng_buf,
             acc_x, acc_c, acc_q,
             red_buf, big_buf, res_buf,
             shared,
             sem0, sem1,
             *, e_per_w, n_last):
    c = lax.axis_index("c")
    s = lax.axis_index("s")
    wid = c * NS + s
    iota = lax.iota(jnp.int32, L)
    lane_base = iota * ACCW
    zeros = jnp.zeros((L,), jnp.float32)
    ones = jnp.ones((L,), jnp.float32)
    n_blk = e_per_w // EB
    sems = (sem0, sem1)

    eb_base = wid * e_per_w

    def start_block(b):
        slot = b % 2
        base = eb_base + b * EB
        return (
            pltpu.async_copy(pq.at[pl.ds(base, EB)],
                             p_buf.at[pl.ds(slot * EB, EB)], sems[slot]),
            pltpu.async_copy(tq.at[pl.ds(base, EB)],
                             t_buf.at[pl.ds(slot * EB, EB)], sems[slot]),
            pltpu.async_copy(e2g.at[pl.ds(base, EB)],
                             g_buf.at[pl.ds(slot * EB, EB)], sems[slot]),
        )

    inflight = {0: start_block(0)}

    def zbody(j, _):
        acc_x[pl.ds(j * L, L)] = zeros
        acc_c[pl.ds(j * L, L)] = zeros
        acc_q[pl.ds(j * L, L)] = zeros
        return 0
    lax.fori_loop(0, (L * ACCW) // L, zbody, 0)

    def node_part(nrows):
        rbase = wid * NR
        pltpu.sync_copy(err.at[pl.ds(rbase, nrows)],
                        ne_buf.at[pl.ds(0, nrows)])
        pltpu.sync_copy(n2g.at[pl.ds(rbase, nrows)],
                        ng_buf.at[pl.ds(0, nrows)])

        def nbody(v, _):
            e2 = ne_buf[pl.ds(v * L, L)]
            ids = ng_buf[pl.ds(v * L, L)]
            idx = lane_base + ids
            plsc.addupdate_scatter(acc_x, [idx], e2)
            plsc.addupdate_scatter(acc_c, [idx], ones)
            return 0
        lax.fori_loop(0, nrows // L, nbody, 0)

    @pl.when(wid < NW - 1)
    def _():
        node_part(NR)

    @pl.when(wid == NW - 1)
    def _():
        node_part(n_last)

    for b in range(n_blk):
        slot = b % 2
        if b + 1 < n_blk:
            inflight[b + 1] = start_block(b + 1)
        for h in inflight.pop(b):
            h.wait()
        sbase = slot * EB

        def ebody(v, _):
            off = sbase + v * (EU * L)
            for u in range(EU):
                o = off + u * L
                p = p_buf[pl.ds(o, L)]
                t = t_buf[pl.ds(o, L)]
                g = g_buf[pl.ds(o, L)]
                d = p - t
                plsc.addupdate_scatter(acc_q, [lane_base + g], d * d)
            return 0
        lax.fori_loop(0, EB // (EU * L), ebody, 0)

    def rbody(j, _):
        col = j * L
        sx = zeros
        sc = zeros
        sq = zeros
        for i in range(L):
            gidx = i * ACCW + col + iota
            sx = sx + plsc.load_gather(acc_x, [gidx])
            sc = sc + plsc.load_gather(acc_c, [gidx])
            sq = sq + plsc.load_gather(acc_q, [gidx])
        red_buf[pl.ds(col, L)] = sx
        red_buf[pl.ds(REDW + col, L)] = sc
        red_buf[pl.ds(2 * REDW + col, L)] = sq
        return 0
    lax.fori_loop(0, REDW // L, rbody, 0)

    pltpu.sync_copy(red_buf, shared.at[pl.ds(s * RED, RED)])
    plsc.subcore_barrier()

    @pl.when(s == 0)
    def _():
        pltpu.sync_copy(shared, big_buf)

        def fbody(j, _):
            col = j * L
            acc = zeros
            for i in range(NS):
                acc = acc + big_buf[pl.ds(i * RED + col, L)]
            res_buf[pl.ds(col, L)] = acc
            return 0
        lax.fori_loop(0, RED // L, fbody, 0)
        pltpu.sync_copy(res_buf, out.at[pl.ds(c * RED, RED)])


def _fin_body(p_ref, o1_ref, o2_ref):
    p = p_ref[...]
    t = p[0] + p[1]
    sx = t[0:1, :]
    cnt = t[1:2, :]
    sq = t[2:3, :]
    rmsd = jnp.sqrt(sx / jnp.maximum(cnt, 1.0))
    o1_ref[0, 0] = jnp.sum(rmsd) * (1.0 / NUM_GRAPHS)
    o2_ref[0, 0] = jnp.sum(jnp.sqrt(sq)) * (1.0 / NUM_GRAPHS)


def kernel(pred_x, pred_q, target_x, target_q, edge2graph, node2graph,
           atom_type, edge_r, edge_p):
    n = node2graph.shape[0]
    e = edge2graph.shape[0]
    assert e % (NW * L) == 0 and (e // NW) % EB == 0
    e_per_w = e // NW

    n_last = n - (NW - 1) * NR
    assert 0 < n_last <= NR and n_last % L == 0

    err2d = pl.pallas_call(
        _err_body,
        grid=(NW,),
        in_specs=[pl.BlockSpec((NR, 3), lambda i: (i, 0)),
                  pl.BlockSpec((NR, 3), lambda i: (i, 0))],
        out_specs=pl.BlockSpec((1, NR), lambda i: (0, i)),
        out_shape=jax.ShapeDtypeStruct((1, NW * NR), jnp.float32),
    )(pred_x, target_x)
    err = err2d.reshape(NW * NR)

    sc_call = pl.kernel(
        functools.partial(_sc_body, e_per_w=e_per_w, n_last=n_last),
        out_type=jax.ShapeDtypeStruct((NC * RED,), jnp.float32),
        mesh=plsc.VectorSubcoreMesh(core_axis_name="c", subcore_axis_name="s"),
        compiler_params=pltpu.CompilerParams(needs_layout_passes=False,
                                             use_tc_tiling_on_sc=False),
        scratch_types=[
            pltpu.VMEM((2 * EB,), jnp.float32),
            pltpu.VMEM((2 * EB,), jnp.float32),
            pltpu.VMEM((2 * EB,), jnp.int32),
            pltpu.VMEM((NR,), jnp.float32),
            pltpu.VMEM((NR,), jnp.int32),
            pltpu.VMEM((L * ACCW,), jnp.float32),
            pltpu.VMEM((L * ACCW,), jnp.float32),
            pltpu.VMEM((L * ACCW,), jnp.float32),
            pltpu.VMEM((RED,), jnp.float32),
            pltpu.VMEM((NS * RED,), jnp.float32),
            pltpu.VMEM((RED,), jnp.float32),
            pltpu.VMEM_SHARED((NS * RED,), jnp.float32),
            pltpu.SemaphoreType.DMA,
            pltpu.SemaphoreType.DMA,
        ],
    )
    partials = sc_call(err, node2graph, pred_q, target_q, edge2graph)
    partials = partials.reshape(NC, 3, REDW)

    r1, r2 = pl.pallas_call(
        _fin_body,
        out_shape=(jax.ShapeDtypeStruct((1, 1), jnp.float32),
                   jax.ShapeDtypeStruct((1, 1), jnp.float32)),
        in_specs=[pl.BlockSpec(memory_space=pltpu.VMEM)],
        out_specs=(pl.BlockSpec(memory_space=pltpu.SMEM),
                   pl.BlockSpec(memory_space=pltpu.SMEM)),
    )(partials)
    return (r1[0, 0], r2[0, 0])
